# baseline (speedup 1.0000x reference)
import functools
import math

import jax
import jax.numpy as jnp
from jax import lax
from jax.experimental import pallas as pl
from jax.experimental.pallas import tpu as pltpu
from jax.experimental.pallas import tpu_sc as plsc

EMBED = 64
SCALE = math.sqrt(EMBED)
CH = 128
NBUF = 2
NC = 2
NS = 16
NW = NC * NS


def _make_sc_kernel(n_chunks: int):
  assert n_chunks % NBUF == 0
  per_w = n_chunks * CH
  mesh = plsc.VectorSubcoreMesh(core_axis_name="c", subcore_axis_name="s")

  @functools.partial(
      pl.kernel,
      out_type=jax.ShapeDtypeStruct((NW * per_w, EMBED), jnp.float32),
      mesh=mesh,
      compiler_params=pltpu.CompilerParams(use_tc_tiling_on_sc=False),
      scratch_types=[
          pltpu.VMEM((n_chunks, CH), jnp.int32),
          pltpu.VMEM((NBUF, CH, EMBED), jnp.float32),
          pltpu.VMEM((NBUF, CH, EMBED), jnp.float32),
          pltpu.VMEM((NBUF, CH, EMBED), jnp.float32),
          [pltpu.SemaphoreType.DMA] * NBUF,
          [pltpu.SemaphoreType.DMA] * NBUF,
          [pltpu.SemaphoreType.DMA] * NBUF,
      ],
  )
  def k(base_hbm, tab_hbm, idx_hbm, out_hbm, idx_v, rows_a, rows_b, out_v,
        sem_a, sem_b, sem_st):
    wid = lax.axis_index("s") * NC + lax.axis_index("c")
    base = wid * per_w
    pltpu.sync_copy(idx_hbm.at[wid], idx_v)

    for p in range(NBUF):
      pltpu.async_copy(base_hbm.at[idx_v.at[p]], rows_a.at[p], sem_a[p])
      pltpu.async_copy(tab_hbm.at[idx_v.at[p]], rows_b.at[p], sem_b[p])

    @pl.loop(0, n_chunks // NBUF)
    def _outer(o):
      for p in range(NBUF):
        g = o * NBUF + p
        ida = idx_v.at[g]
        pltpu.make_async_copy(base_hbm.at[ida], rows_a.at[p], sem_a[p]).wait()
        pltpu.make_async_copy(tab_hbm.at[ida], rows_b.at[p], sem_b[p]).wait()

        @pl.when(o > 0)
        def _():
          pltpu.make_async_copy(
              out_v.at[p], out_hbm.at[pl.ds(base, CH)], sem_st[p]).wait()

        @pl.loop(0, CH, unroll=4)
        def _row(j):
          for d in range(EMBED // 16):
            sl = pl.ds(d * 16, 16)
            out_v[p, j, sl] = rows_a[p, j, sl] + SCALE * rows_b[p, j, sl]

        pltpu.async_copy(
            out_v.at[p], out_hbm.at[pl.ds(base + g * CH, CH)], sem_st[p])

        @pl.when(g + NBUF < n_chunks)
        def _():
          idn = idx_v.at[g + NBUF]
          pltpu.async_copy(base_hbm.at[idn], rows_a.at[p], sem_a[p])
          pltpu.async_copy(tab_hbm.at[idn], rows_b.at[p], sem_b[p])

    for p in range(NBUF):
      pltpu.make_async_copy(
          out_v.at[p], out_hbm.at[pl.ds(base, CH)], sem_st[p]).wait()

  return k


def kernel(tokens, base_table, table):
  shape = tokens.shape
  n = tokens.size
  idx = tokens.reshape(-1).astype(jnp.int32)
  n_chunks = -(-n // (NW * CH * NBUF)) * NBUF
  n_pad = n_chunks * CH * NW
  if n_pad != n:
    idx = jnp.pad(idx, (0, n_pad - n))
  idx3 = idx.reshape(NW, n_chunks, CH)
  out = _make_sc_kernel(n_chunks)(base_table, table, idx3)
  if n_pad != n:
    out = out[:n]
  return out.reshape(*shape, EMBED)


# speedup vs baseline: 1.0892x; 1.0892x over previous
import functools
import math

import jax
import jax.numpy as jnp
from jax import lax
from jax.experimental import pallas as pl
from jax.experimental.pallas import tpu as pltpu
from jax.experimental.pallas import tpu_sc as plsc

EMBED = 64
SCALE = math.sqrt(EMBED)
CH = 256
NBUF = 2
NC = 2
NS = 16
NW = NC * NS


def _make_sc_kernel(n_chunks: int):
  assert n_chunks % NBUF == 0
  per_w = n_chunks * CH
  mesh = plsc.VectorSubcoreMesh(core_axis_name="c", subcore_axis_name="s")

  @functools.partial(
      pl.kernel,
      out_type=jax.ShapeDtypeStruct((NW * per_w, EMBED), jnp.float32),
      mesh=mesh,
      compiler_params=pltpu.CompilerParams(use_tc_tiling_on_sc=False),
      scratch_types=[
          pltpu.VMEM((per_w,), jnp.int32),
          pltpu.VMEM((NBUF, CH, EMBED), jnp.float32),
          pltpu.VMEM((NBUF, CH, EMBED), jnp.float32),
          pltpu.VMEM((NBUF, CH, EMBED), jnp.float32),
          [pltpu.SemaphoreType.DMA] * NBUF,
          [pltpu.SemaphoreType.DMA] * NBUF,
          [pltpu.SemaphoreType.DMA] * NBUF,
      ],
  )
  def k(base_hbm, tab_hbm, idx_hbm, out_hbm, idx_v, rows_a, rows_b, out_v,
        sem_a, sem_b, sem_st):
    wid = lax.axis_index("s") * NC + lax.axis_index("c")
    base = wid * per_w
    pltpu.sync_copy(idx_hbm.at[pl.ds(base, per_w)], idx_v)

    for p in range(NBUF):
      ida = idx_v.at[pl.ds(p * CH, CH)]
      pltpu.async_copy(base_hbm.at[ida], rows_a.at[p], sem_a[p])
      pltpu.async_copy(tab_hbm.at[ida], rows_b.at[p], sem_b[p])

    @pl.loop(0, n_chunks // NBUF)
    def _outer(o):
      for p in range(NBUF):
        g = o * NBUF + p
        ida = idx_v.at[pl.ds(g * CH, CH)]
        pltpu.make_async_copy(base_hbm.at[ida], rows_a.at[p], sem_a[p]).wait()
        pltpu.make_async_copy(tab_hbm.at[ida], rows_b.at[p], sem_b[p]).wait()

        @pl.when(o > 0)
        def _():
          pltpu.make_async_copy(
              out_v.at[p], out_hbm.at[pl.ds(base, CH)], sem_st[p]).wait()

        @pl.loop(0, CH, unroll=4)
        def _row(j):
          for d in range(EMBED // 16):
            sl = pl.ds(d * 16, 16)
            out_v[p, j, sl] = rows_a[p, j, sl] + SCALE * rows_b[p, j, sl]

        pltpu.async_copy(
            out_v.at[p], out_hbm.at[pl.ds(base + g * CH, CH)], sem_st[p])

        @pl.when(g + NBUF < n_chunks)
        def _():
          idn = idx_v.at[pl.ds((g + NBUF) * CH, CH)]
          pltpu.async_copy(base_hbm.at[idn], rows_a.at[p], sem_a[p])
          pltpu.async_copy(tab_hbm.at[idn], rows_b.at[p], sem_b[p])

    for p in range(NBUF):
      pltpu.make_async_copy(
          out_v.at[p], out_hbm.at[pl.ds(base, CH)], sem_st[p]).wait()

  return k


def kernel(tokens, base_table, table):
  shape = tokens.shape
  n = tokens.size
  idx = tokens.reshape(-1).astype(jnp.int32)
  n_chunks = -(-n // (NW * CH * NBUF)) * NBUF
  n_pad = n_chunks * CH * NW
  if n_pad != n:
    idx = jnp.pad(idx, (0, n_pad - n))
  out = _make_sc_kernel(n_chunks)(base_table, table, idx)
  if n_pad != n:
    out = out[:n]
  return out.reshape(*shape, EMBED)


# speedup vs baseline: 1.1622x; 1.0670x over previous
import functools
import math

import jax
import jax.numpy as jnp
from jax import lax
from jax.experimental import pallas as pl
from jax.experimental.pallas import tpu as pltpu
from jax.experimental.pallas import tpu_sc as plsc

EMBED = 64
SCALE = math.sqrt(EMBED)
CH = 160
NBUF = 2
NC = 2
NS = 16
NW = NC * NS


def _make_sc_kernel(n_chunks: int):
  assert n_chunks % NBUF == 0
  per_w = n_chunks * CH
  mesh = plsc.VectorSubcoreMesh(core_axis_name="c", subcore_axis_name="s")

  @functools.partial(
      pl.kernel,
      out_type=jax.ShapeDtypeStruct((NW * per_w, 2 * EMBED), jnp.float32),
      mesh=mesh,
      compiler_params=pltpu.CompilerParams(use_tc_tiling_on_sc=False),
      scratch_types=[
          pltpu.VMEM((per_w,), jnp.int32),
          pltpu.VMEM((NBUF, CH, EMBED), jnp.float32),
          pltpu.VMEM((NBUF, CH, EMBED), jnp.float32),
          pltpu.VMEM((NBUF, CH, 2 * EMBED), jnp.float32),
          [pltpu.SemaphoreType.DMA] * NBUF,
          [pltpu.SemaphoreType.DMA] * NBUF,
          [pltpu.SemaphoreType.DMA] * NBUF,
      ],
  )
  def k(base_hbm, tab_hbm, idx_hbm, out_hbm, idx_v, rows_a, rows_b, out_v,
        sem_a, sem_b, sem_st):
    wid = lax.axis_index("s") * NC + lax.axis_index("c")
    base = wid * per_w
    pltpu.sync_copy(idx_hbm.at[pl.ds(base, per_w)], idx_v)

    for p in range(NBUF):
      ida = idx_v.at[pl.ds(p * CH, CH)]
      pltpu.async_copy(base_hbm.at[ida], rows_a.at[p], sem_a[p])
      pltpu.async_copy(tab_hbm.at[ida], rows_b.at[p], sem_b[p])

    @pl.loop(0, n_chunks // NBUF)
    def _outer(o):
      for p in range(NBUF):
        g = o * NBUF + p
        ida = idx_v.at[pl.ds(g * CH, CH)]
        pltpu.make_async_copy(base_hbm.at[ida], rows_a.at[p], sem_a[p]).wait()
        pltpu.make_async_copy(tab_hbm.at[ida], rows_b.at[p], sem_b[p]).wait()

        @pl.when(o > 0)
        def _():
          pltpu.make_async_copy(
              out_v.at[p], out_hbm.at[pl.ds(base, CH)], sem_st[p]).wait()

        @pl.loop(0, CH, unroll=4)
        def _row(j):
          for d in range(EMBED // 16):
            sl = pl.ds(d * 16, 16)
            out_v[p, j, sl] = rows_a[p, j, sl] + SCALE * rows_b[p, j, sl]

        pltpu.async_copy(
            out_v.at[p], out_hbm.at[pl.ds(base + g * CH, CH)], sem_st[p])

        @pl.when(g + NBUF < n_chunks)
        def _():
          idn = idx_v.at[pl.ds((g + NBUF) * CH, CH)]
          pltpu.async_copy(base_hbm.at[idn], rows_a.at[p], sem_a[p])
          pltpu.async_copy(tab_hbm.at[idn], rows_b.at[p], sem_b[p])

    for p in range(NBUF):
      pltpu.make_async_copy(
          out_v.at[p], out_hbm.at[pl.ds(base, CH)], sem_st[p]).wait()

  return k


def kernel(tokens, base_table, table):
  shape = tokens.shape
  n = tokens.size
  idx = tokens.reshape(-1).astype(jnp.int32)
  n_chunks = -(-n // (NW * CH * NBUF)) * NBUF
  n_pad = n_chunks * CH * NW
  if n_pad != n:
    idx = jnp.pad(idx, (0, n_pad - n))
  out = _make_sc_kernel(n_chunks)(base_table, table, idx)
  out = out[:n, :EMBED]
  return out.reshape(*shape, EMBED)


# speedup vs baseline: 1.1635x; 1.0011x over previous
import functools
import math

import jax
import jax.numpy as jnp
from jax import lax
from jax.experimental import pallas as pl
from jax.experimental.pallas import tpu as pltpu
from jax.experimental.pallas import tpu_sc as plsc

EMBED = 64
SCALE = math.sqrt(EMBED)
CH = 160
NBUF = 2
NC = 2
NS = 16
NW = NC * NS


def _make_sc_kernel(n_chunks: int):
  assert n_chunks % NBUF == 0
  per_w = n_chunks * CH
  mesh = plsc.VectorSubcoreMesh(core_axis_name="c", subcore_axis_name="s")

  @functools.partial(
      pl.kernel,
      out_type=jax.ShapeDtypeStruct((NW * per_w, 2 * EMBED), jnp.float32),
      mesh=mesh,
      compiler_params=pltpu.CompilerParams(use_tc_tiling_on_sc=False),
      scratch_types=[
          pltpu.VMEM((per_w,), jnp.int32),
          pltpu.VMEM((NBUF, CH, EMBED), jnp.float32),
          pltpu.VMEM((NBUF, CH, EMBED), jnp.float32),
          pltpu.VMEM((NBUF, CH, 2 * EMBED), jnp.float32),
          [pltpu.SemaphoreType.DMA] * NBUF,
          [pltpu.SemaphoreType.DMA] * NBUF,
          [pltpu.SemaphoreType.DMA] * NBUF,
      ],
  )
  def k(base_hbm, tab_hbm, idx_hbm, out_hbm, idx_v, rows_a, rows_b, out_v,
        sem_a, sem_b, sem_st):
    wid = lax.axis_index("s") * NC + lax.axis_index("c")
    base = wid * per_w
    pltpu.sync_copy(idx_hbm.at[pl.ds(base, per_w)], idx_v)

    for p in range(NBUF):
      ida = idx_v.at[pl.ds(p * CH, CH)]
      pltpu.async_copy(base_hbm.at[ida], rows_a.at[p], sem_a[p])
      pltpu.async_copy(tab_hbm.at[ida], rows_b.at[p], sem_b[p])

    @pl.loop(0, n_chunks // NBUF)
    def _outer(o):
      for p in range(NBUF):
        g = o * NBUF + p
        ida = idx_v.at[pl.ds(g * CH, CH)]
        pltpu.make_async_copy(base_hbm.at[ida], rows_a.at[p], sem_a[p]).wait()
        pltpu.make_async_copy(tab_hbm.at[ida], rows_b.at[p], sem_b[p]).wait()

        @pl.when(o > 0)
        def _():
          pltpu.make_async_copy(
              out_v.at[p], out_hbm.at[pl.ds(base, CH)], sem_st[p]).wait()

        @pl.loop(0, CH, unroll=4)
        def _row(j):
          for d in range(EMBED // 16):
            sl = pl.ds(d * 16, 16)
            out_v[p, j, sl] = rows_a[p, j, sl] + SCALE * rows_b[p, j, sl]

        pltpu.async_copy(
            out_v.at[p], out_hbm.at[pl.ds(base + g * CH, CH)], sem_st[p])

        @pl.when(g + NBUF < n_chunks)
        def _():
          idn = idx_v.at[pl.ds((g + NBUF) * CH, CH)]
          pltpu.async_copy(base_hbm.at[idn], rows_a.at[p], sem_a[p])
          pltpu.async_copy(tab_hbm.at[idn], rows_b.at[p], sem_b[p])

    for p in range(NBUF):
      pltpu.make_async_copy(
          out_v.at[p], out_hbm.at[pl.ds(base, CH)], sem_st[p]).wait()

  return k


def kernel(tokens, base_table, table):
  shape = tokens.shape
  n = tokens.size
  idx = tokens.reshape(-1).astype(jnp.int32)
  n_chunks = -(-n // (NW * CH * NBUF)) * NBUF
  n_pad = n_chunks * CH * NW
  if n_pad != n:
    idx = jnp.pad(idx, (0, n_pad - n))
  out = _make_sc_kernel(n_chunks)(base_table, table, idx)
  out = out[:n].reshape(*shape, 2 * EMBED)
  return out[..., :EMBED]


# speedup vs baseline: 1.1660x; 1.0022x over previous
import functools
import math

import jax
import jax.numpy as jnp
from jax import lax
from jax.experimental import pallas as pl
from jax.experimental.pallas import tpu as pltpu
from jax.experimental.pallas import tpu_sc as plsc

EMBED = 64
SCALE = math.sqrt(EMBED)
CH = 200
NBUF = 2
NC = 2
NS = 16
NW = NC * NS


def _make_sc_kernel(n_chunks: int):
  assert n_chunks % NBUF == 0
  per_w = n_chunks * CH
  mesh = plsc.VectorSubcoreMesh(core_axis_name="c", subcore_axis_name="s")

  @functools.partial(
      pl.kernel,
      out_type=jax.ShapeDtypeStruct((NW * per_w, 2 * EMBED), jnp.float32),
      mesh=mesh,
      compiler_params=pltpu.CompilerParams(use_tc_tiling_on_sc=False),
      scratch_types=[
          pltpu.VMEM((per_w,), jnp.int32),
          pltpu.VMEM((NBUF, CH, EMBED), jnp.float32),
          pltpu.VMEM((NBUF, CH, EMBED), jnp.float32),
          pltpu.VMEM((NBUF, CH, 2 * EMBED), jnp.float32),
          [pltpu.SemaphoreType.DMA] * NBUF,
          [pltpu.SemaphoreType.DMA] * NBUF,
          [pltpu.SemaphoreType.DMA] * NBUF,
      ],
  )
  def k(base_hbm, tab_hbm, idx_hbm, out_hbm, idx_v, rows_a, rows_b, out_v,
        sem_a, sem_b, sem_st):
    wid = lax.axis_index("s") * NC + lax.axis_index("c")
    base = wid * per_w
    pltpu.sync_copy(idx_hbm.at[pl.ds(base, per_w)], idx_v)

    for p in range(NBUF):
      ida = idx_v.at[pl.ds(p * CH, CH)]
      pltpu.async_copy(base_hbm.at[ida], rows_a.at[p], sem_a[p])
      pltpu.async_copy(tab_hbm.at[ida], rows_b.at[p], sem_b[p])

    @pl.loop(0, n_chunks // NBUF)
    def _outer(o):
      for p in range(NBUF):
        g = o * NBUF + p
        ida = idx_v.at[pl.ds(g * CH, CH)]
        pltpu.make_async_copy(base_hbm.at[ida], rows_a.at[p], sem_a[p]).wait()
        pltpu.make_async_copy(tab_hbm.at[ida], rows_b.at[p], sem_b[p]).wait()

        @pl.when(o > 0)
        def _():
          pltpu.make_async_copy(
              out_v.at[p], out_hbm.at[pl.ds(base, CH)], sem_st[p]).wait()

        @pl.loop(0, CH, unroll=4)
        def _row(j):
          for d in range(EMBED // 16):
            sl = pl.ds(d * 16, 16)
            out_v[p, j, sl] = rows_a[p, j, sl] + SCALE * rows_b[p, j, sl]

        pltpu.async_copy(
            out_v.at[p], out_hbm.at[pl.ds(base + g * CH, CH)], sem_st[p])

        @pl.when(g + NBUF < n_chunks)
        def _():
          idn = idx_v.at[pl.ds((g + NBUF) * CH, CH)]
          pltpu.async_copy(base_hbm.at[idn], rows_a.at[p], sem_a[p])
          pltpu.async_copy(tab_hbm.at[idn], rows_b.at[p], sem_b[p])

    for p in range(NBUF):
      pltpu.make_async_copy(
          out_v.at[p], out_hbm.at[pl.ds(base, CH)], sem_st[p]).wait()

  return k


def kernel(tokens, base_table, table):
  shape = tokens.shape
  n = tokens.size
  idx = tokens.reshape(-1).astype(jnp.int32)
  n_chunks = -(-n // (NW * CH * NBUF)) * NBUF
  n_pad = n_chunks * CH * NW
  if n_pad != n:
    idx = jnp.pad(idx, (0, n_pad - n))
  out = _make_sc_kernel(n_chunks)(base_table, table, idx)
  out = out[:n].reshape(*shape, 2 * EMBED)
  return out[..., :EMBED]
